# Initial kernel scaffold; baseline (speedup 1.0000x reference)
import jax
import jax.numpy as jnp
from jax.experimental import pallas as pl


def kernel(x_bld, pos_table):
    raise NotImplementedError("write your pallas kernel here")


# speedup vs baseline: 1.5014x; 1.5014x over previous
import jax
import jax.numpy as jnp
from jax.experimental import pallas as pl

_TL = 512


def _body(x_ref, p_ref, o_ref):
    o_ref[...] = x_ref[...] + p_ref[...]


def kernel(x_bld, pos_table):
    B, L, D = x_bld.shape
    return pl.pallas_call(
        _body,
        grid=(L // _TL, B),
        in_specs=[
            pl.BlockSpec((1, _TL, D), lambda l, b: (b, l, 0)),
            pl.BlockSpec((_TL, D), lambda l, b: (l, 0)),
        ],
        out_specs=pl.BlockSpec((1, _TL, D), lambda l, b: (b, l, 0)),
        out_shape=jax.ShapeDtypeStruct(x_bld.shape, x_bld.dtype),
    )(x_bld, pos_table)


# speedup vs baseline: 1.7425x; 1.1606x over previous
import jax
import jax.numpy as jnp
from jax.experimental import pallas as pl

_TL = 2048


def _body(x_ref, p_ref, o_ref):
    o_ref[...] = x_ref[...] + p_ref[...]


def kernel(x_bld, pos_table):
    B, L, D = x_bld.shape
    return pl.pallas_call(
        _body,
        grid=(L // _TL, B),
        in_specs=[
            pl.BlockSpec((1, _TL, D), lambda l, b: (b, l, 0)),
            pl.BlockSpec((_TL, D), lambda l, b: (l, 0)),
        ],
        out_specs=pl.BlockSpec((1, _TL, D), lambda l, b: (b, l, 0)),
        out_shape=jax.ShapeDtypeStruct(x_bld.shape, x_bld.dtype),
    )(x_bld, pos_table)
